# Initial kernel scaffold; baseline (speedup 1.0000x reference)
import jax
import jax.numpy as jnp
from jax.experimental import pallas as pl


def kernel(h_v, h_w, e_vw, edge_matrix):
    raise NotImplementedError("write your pallas kernel here")


# speedup vs baseline: 1.3401x; 1.3401x over previous
import functools

import jax
import jax.numpy as jnp
from jax.experimental import pallas as pl

N_LABELS = 4
IN_SIZE = 128
OUT_SIZE = 128
BLK = 1280


def _ggnn_body(e_ref, x_ref, wt_ref, o_ref):
    x = x_ref[...]
    p = jnp.dot(x, wt_ref[...], preferred_element_type=jnp.float32)
    e = e_ref[...]
    acc = jnp.where(e == 0, p[:, 0:OUT_SIZE], 0.0)
    for lab in range(1, N_LABELS):
        acc = acc + jnp.where(e == lab, p[:, lab * OUT_SIZE:(lab + 1) * OUT_SIZE], 0.0)
    o_ref[...] = acc


@functools.partial(jax.jit, static_argnames=("interpret",))
def kernel(h_v, h_w, e_vw, edge_matrix, interpret=False):
    del h_v
    b, n, _ = h_w.shape
    rows = b * n
    x = h_w.reshape(rows, IN_SIZE)
    e = e_vw.reshape(rows, 1)
    wt = jnp.transpose(edge_matrix, (2, 0, 1)).reshape(IN_SIZE, N_LABELS * OUT_SIZE)

    grid = rows // BLK
    out = pl.pallas_call(
        _ggnn_body,
        grid=(grid,),
        in_specs=[
            pl.BlockSpec((BLK, 1), lambda i: (i, 0)),
            pl.BlockSpec((BLK, IN_SIZE), lambda i: (i, 0)),
            pl.BlockSpec((IN_SIZE, N_LABELS * OUT_SIZE), lambda i: (0, 0)),
        ],
        out_specs=pl.BlockSpec((BLK, OUT_SIZE), lambda i: (i, 0)),
        out_shape=jax.ShapeDtypeStruct((rows, OUT_SIZE), h_w.dtype),
        interpret=interpret,
    )(e, x, wt)
    return out.reshape(b, n, OUT_SIZE)
